# Initial kernel scaffold; baseline (speedup 1.0000x reference)
import jax
import jax.numpy as jnp
from jax.experimental import pallas as pl


def kernel(query, reference_points, value, value_spatial_shapes, Wv, bv, Wo, bo, Wa, ba, Wout, bout):
    raise NotImplementedError("write your pallas kernel here")


# speedup vs baseline: 123.8864x; 123.8864x over previous
import functools

import jax
import jax.numpy as jnp
import numpy as np
from jax import lax
from jax.experimental import pallas as pl
from jax.experimental.pallas import tpu as pltpu
from jax.experimental.pallas import tpu_sc as plsc

B = 8
LQ = 300
E = 256
H = 8
HD = 32
P = 16
SPATIAL = ((128, 128), (64, 64), (32, 32), (16, 16))
LV = sum(h * w for h, w in SPATIAL)
NCORN = 4 * P
N4 = B * H * NCORN * LQ
NW = 32
ROWS_PER_W = N4 // NW
CHUNK = 128
NCHUNK = ROWS_PER_W // CHUNK
LV_BLK = 1280
OFFSET_MUL = 0.125


def _vproj_body(v_ref, w_ref, b_ref, o_ref):
    o_ref[0] = (
        jnp.dot(v_ref[0], w_ref[...], preferred_element_type=jnp.float32)
        + b_ref[...]
    )


def _idx_body(qt_ref, rpt_ref, wa_ref, ba_ref, wox_ref, box_ref, woy_ref,
              boy_ref, c_ref, idx_ref, wt_ref):
    b = pl.program_id(0)
    h = pl.program_id(1)
    q = qt_ref[0]
    a = jnp.dot(wa_ref[0], q, preferred_element_type=jnp.float32) + ba_ref[0]
    m = jnp.max(a, axis=0, keepdims=True)
    e = jnp.exp(a - m)
    aw = e / jnp.sum(e, axis=0, keepdims=True)
    sx = jnp.dot(wox_ref[0], q, preferred_element_type=jnp.float32) + box_ref[0]
    sy = jnp.dot(woy_ref[0], q, preferred_element_type=jnp.float32) + boy_ref[0]
    refx = rpt_ref[0, 0:1, :]
    refy = rpt_ref[0, 1:2, :]
    refw = rpt_ref[0, 2:3, :]
    refh = rpt_ref[0, 3:4, :]
    wc = c_ref[:, 0:1]
    hc = c_ref[:, 1:2]
    sv = c_ref[:, 2:3]
    ix = (refx + sx * OFFSET_MUL * refw) * wc - 0.5
    iy = (refy + sy * OFFSET_MUL * refh) * hc - 0.5
    x0 = jnp.floor(ix)
    y0 = jnp.floor(iy)
    fx = ix - x0
    fy = iy - y0
    base = b * (LV * H) + h
    for ci, (dx, dy) in enumerate(((0.0, 0.0), (1.0, 0.0), (0.0, 1.0),
                                   (1.0, 1.0))):
        xn = x0 + dx
        yn = y0 + dy
        valid = ((xn >= 0.0) & (xn <= wc - 1.0)
                 & (yn >= 0.0) & (yn <= hc - 1.0))
        xc = jnp.clip(xn, 0.0, wc - 1.0)
        yc = jnp.clip(yn, 0.0, hc - 1.0)
        pos = sv + yc * wc + xc
        r = pos.astype(jnp.int32) * H + base
        wx = fx if dx else 1.0 - fx
        wy = fy if dy else 1.0 - fy
        wgt = jnp.where(valid, wx * wy * aw, 0.0)
        idx_ref[0, 0, ci * P:(ci + 1) * P, :] = r
        wt_ref[0, 0, ci * P:(ci + 1) * P, :] = wgt


def _gather_body(table_ref, idx_ref, out_ref, idx_v, buf_v, sem):
    wid = lax.axis_index("s") * 2 + lax.axis_index("c")
    pltpu.sync_copy(idx_ref.at[wid], idx_v)

    def step(g, carry):
        pltpu.async_copy(table_ref.at[idx_v.at[g]], buf_v, sem).wait()
        pltpu.sync_copy(
            buf_v, out_ref.at[pl.ds(wid * ROWS_PER_W + g * CHUNK, CHUNK)])
        return carry

    lax.fori_loop(0, NCHUNK, step, 0)


def _reduce_body(g_ref, w_ref, wo_ref, bo_ref, o_ref):
    h = pl.program_id(1)
    g = g_ref[0, 0]
    w4 = w_ref[0, 0]
    hs = None
    for c in range(4):
        part = jnp.sum(g[:, :, c * HD:(c + 1) * HD] * w4[c][:, :, None],
                       axis=1)
        hs = part if hs is None else hs + part
    part = jnp.dot(hs, wo_ref[0], preferred_element_type=jnp.float32)

    @pl.when(h == 0)
    def _():
        o_ref[0] = part + bo_ref[...]

    @pl.when(h != 0)
    def _():
        o_ref[0] = o_ref[0] + part


_LEVEL_CONSTS = np.zeros((P, 3), np.float32)
_sv = 0
for _l, (_h, _w) in enumerate(SPATIAL):
    _LEVEL_CONSTS[_l * 4:(_l + 1) * 4, 0] = _w
    _LEVEL_CONSTS[_l * 4:(_l + 1) * 4, 1] = _h
    _LEVEL_CONSTS[_l * 4:(_l + 1) * 4, 2] = _sv
    _sv += _h * _w


def kernel(query, reference_points, value, value_spatial_shapes, Wv, bv, Wo,
           bo, Wa, ba, Wout, bout):
    del value_spatial_shapes
    f32 = jnp.float32

    qt = query.transpose(0, 2, 1)
    rpt = reference_points[:, :, 0, :].transpose(0, 2, 1)
    wo4 = Wo.reshape(H, P, 2, E)
    wox, woy = wo4[:, :, 0, :], wo4[:, :, 1, :]
    bo4 = bo.reshape(H, P, 2)
    box, boy = bo4[:, :, 0][:, :, None], bo4[:, :, 1][:, :, None]
    wa3 = Wa.reshape(H, P, E)
    ba3 = ba.reshape(H, P, 1)
    consts = jnp.asarray(_LEVEL_CONSTS)
    wvt = Wv.T
    bv2 = bv.reshape(1, E)
    wot = Wout.T
    bout2 = bout.reshape(1, E)

    nblk = LV // LV_BLK
    table = pl.pallas_call(
        _vproj_body,
        grid=(B, nblk),
        in_specs=[
            pl.BlockSpec((1, LV_BLK, E), lambda b, i: (b, i, 0)),
            pl.BlockSpec((E, E), lambda b, i: (0, 0)),
            pl.BlockSpec((1, E), lambda b, i: (0, 0)),
        ],
        out_specs=pl.BlockSpec((1, LV_BLK, E), lambda b, i: (b, i, 0)),
        out_shape=jax.ShapeDtypeStruct((B, LV, E), f32),
    )(value, wvt, bv2)

    idx, wts = pl.pallas_call(
        _idx_body,
        grid=(B, H),
        in_specs=[
            pl.BlockSpec((1, E, LQ), lambda b, h: (b, 0, 0)),
            pl.BlockSpec((1, 4, LQ), lambda b, h: (b, 0, 0)),
            pl.BlockSpec((1, P, E), lambda b, h: (h, 0, 0)),
            pl.BlockSpec((1, P, 1), lambda b, h: (h, 0, 0)),
            pl.BlockSpec((1, P, E), lambda b, h: (h, 0, 0)),
            pl.BlockSpec((1, P, 1), lambda b, h: (h, 0, 0)),
            pl.BlockSpec((1, P, E), lambda b, h: (h, 0, 0)),
            pl.BlockSpec((1, P, 1), lambda b, h: (h, 0, 0)),
            pl.BlockSpec((P, 3), lambda b, h: (0, 0)),
        ],
        out_specs=[
            pl.BlockSpec((1, 1, NCORN, LQ), lambda b, h: (b, h, 0, 0)),
            pl.BlockSpec((1, 1, NCORN, LQ), lambda b, h: (b, h, 0, 0)),
        ],
        out_shape=[
            jax.ShapeDtypeStruct((B, H, NCORN, LQ), jnp.int32),
            jax.ShapeDtypeStruct((B, H, NCORN, LQ), f32),
        ],
    )(qt, rpt, wa3, ba3, wox, box, woy, boy, consts)

    table_rows = table.reshape(B * LV * H, HD)
    idx5 = idx.reshape(B, H, 4, P, LQ).transpose(0, 1, 4, 3, 2)
    idx_flat = idx5.reshape(NW, NCHUNK, CHUNK)
    w5 = wts.reshape(B, H, 4, P, LQ).transpose(0, 1, 2, 4, 3)
    mesh = plsc.VectorSubcoreMesh(core_axis_name="c", subcore_axis_name="s")
    gathered = pl.kernel(
        _gather_body,
        out_type=jax.ShapeDtypeStruct((N4, HD), f32),
        mesh=mesh,
        scratch_types=[
            pltpu.VMEM((NCHUNK, CHUNK), jnp.int32),
            pltpu.VMEM((CHUNK, HD), f32),
            pltpu.SemaphoreType.DMA,
        ],
        compiler_params=pltpu.CompilerParams(use_tc_tiling_on_sc=False),
    )(table_rows, idx_flat)

    g5 = gathered.reshape(B, H, LQ, P, 4 * HD)
    wot3 = wot.reshape(H, HD, E)
    out = pl.pallas_call(
        _reduce_body,
        grid=(B, H),
        in_specs=[
            pl.BlockSpec((1, 1, LQ, P, 4 * HD), lambda b, h: (b, h, 0, 0, 0)),
            pl.BlockSpec((1, 1, 4, LQ, P), lambda b, h: (b, h, 0, 0, 0)),
            pl.BlockSpec((1, HD, E), lambda b, h: (h, 0, 0)),
            pl.BlockSpec((1, E), lambda b, h: (0, 0)),
        ],
        out_specs=pl.BlockSpec((1, LQ, E), lambda b, h: (b, 0, 0)),
        out_shape=jax.ShapeDtypeStruct((B, LQ, E), f32),
    )(g5, w5, wot3, bout2)
    return out


# speedup vs baseline: 130.0956x; 1.0501x over previous
import functools

import jax
import jax.numpy as jnp
import numpy as np
from jax import lax
from jax.experimental import pallas as pl
from jax.experimental.pallas import tpu as pltpu
from jax.experimental.pallas import tpu_sc as plsc

B = 8
LQ = 300
E = 256
H = 8
HD = 32
P = 16
SPATIAL = ((128, 128), (64, 64), (32, 32), (16, 16))
LV = sum(h * w for h, w in SPATIAL)
NCORN = 4 * P
N4 = B * H * NCORN * LQ
NW = 32
ROWS_PER_W = N4 // NW
CHUNK = 128
NCHUNK = ROWS_PER_W // CHUNK
LV_BLK = 1280
OFFSET_MUL = 0.125


def _vproj_body(v_ref, w_ref, b_ref, o_ref):
    o_ref[0] = (
        jnp.dot(v_ref[0], w_ref[...], preferred_element_type=jnp.float32)
        + b_ref[...]
    )


def _idx_body(qt_ref, rpt_ref, wa_ref, ba_ref, wox_ref, box_ref, woy_ref,
              boy_ref, c_ref, idx_ref, wt_ref):
    b = pl.program_id(0)
    h = pl.program_id(1)
    q = qt_ref[0]
    a = jnp.dot(wa_ref[0], q, preferred_element_type=jnp.float32) + ba_ref[0]
    m = jnp.max(a, axis=0, keepdims=True)
    e = jnp.exp(a - m)
    aw = e / jnp.sum(e, axis=0, keepdims=True)
    sx = jnp.dot(wox_ref[0], q, preferred_element_type=jnp.float32) + box_ref[0]
    sy = jnp.dot(woy_ref[0], q, preferred_element_type=jnp.float32) + boy_ref[0]
    refx = rpt_ref[0, 0:1, :]
    refy = rpt_ref[0, 1:2, :]
    refw = rpt_ref[0, 2:3, :]
    refh = rpt_ref[0, 3:4, :]
    wc = c_ref[:, 0:1]
    hc = c_ref[:, 1:2]
    sv = c_ref[:, 2:3]
    ix = (refx + sx * OFFSET_MUL * refw) * wc - 0.5
    iy = (refy + sy * OFFSET_MUL * refh) * hc - 0.5
    x0 = jnp.floor(ix)
    y0 = jnp.floor(iy)
    fx = ix - x0
    fy = iy - y0
    base = b * (LV * H) + h
    for ci, (dx, dy) in enumerate(((0.0, 0.0), (1.0, 0.0), (0.0, 1.0),
                                   (1.0, 1.0))):
        xn = x0 + dx
        yn = y0 + dy
        valid = ((xn >= 0.0) & (xn <= wc - 1.0)
                 & (yn >= 0.0) & (yn <= hc - 1.0))
        xc = jnp.clip(xn, 0.0, wc - 1.0)
        yc = jnp.clip(yn, 0.0, hc - 1.0)
        pos = sv + yc * wc + xc
        r = pos.astype(jnp.int32) * H + base
        wx = fx if dx else 1.0 - fx
        wy = fy if dy else 1.0 - fy
        wgt = jnp.where(valid, wx * wy * aw, 0.0)
        idx_ref[0, 0, ci * P:(ci + 1) * P, :] = r
        wt_ref[0, 0, ci * P:(ci + 1) * P, :] = wgt


def _gather_body(table_ref, idx_ref, out_ref, idx_v, buf_a, buf_b, sem_a,
                 sem_b):
    wid = lax.axis_index("s") * 2 + lax.axis_index("c")
    pltpu.sync_copy(idx_ref.at[wid], idx_v)
    base = wid * ROWS_PER_W
    pltpu.async_copy(table_ref.at[idx_v.at[0]], buf_a, sem_a)

    def pair(g2, carry):
        ga = 2 * g2
        pltpu.make_async_copy(table_ref.at[idx_v.at[ga]], buf_a, sem_a).wait()
        pltpu.async_copy(table_ref.at[idx_v.at[ga + 1]], buf_b, sem_b)
        pltpu.sync_copy(buf_a, out_ref.at[pl.ds(base + ga * CHUNK, CHUNK)])
        pltpu.make_async_copy(
            table_ref.at[idx_v.at[ga + 1]], buf_b, sem_b).wait()

        @pl.when(g2 < NCHUNK // 2 - 1)
        def _():
            pltpu.async_copy(table_ref.at[idx_v.at[ga + 2]], buf_a, sem_a)

        pltpu.sync_copy(
            buf_b, out_ref.at[pl.ds(base + (ga + 1) * CHUNK, CHUNK)])
        return carry

    lax.fori_loop(0, NCHUNK // 2, pair, 0)


def _reduce_body(g_ref, w_ref, wo_ref, bo_ref, o_ref):
    h = pl.program_id(1)
    g = g_ref[0, 0]
    w4 = w_ref[0, 0]
    hs = None
    for c in range(4):
        part = jnp.sum(g[:, :, c * HD:(c + 1) * HD] * w4[c][:, :, None],
                       axis=1)
        hs = part if hs is None else hs + part
    part = jnp.dot(hs, wo_ref[0], preferred_element_type=jnp.float32)

    @pl.when(h == 0)
    def _():
        o_ref[0] = part + bo_ref[...]

    @pl.when(h != 0)
    def _():
        o_ref[0] = o_ref[0] + part


_LEVEL_CONSTS = np.zeros((P, 3), np.float32)
_sv = 0
for _l, (_h, _w) in enumerate(SPATIAL):
    _LEVEL_CONSTS[_l * 4:(_l + 1) * 4, 0] = _w
    _LEVEL_CONSTS[_l * 4:(_l + 1) * 4, 1] = _h
    _LEVEL_CONSTS[_l * 4:(_l + 1) * 4, 2] = _sv
    _sv += _h * _w


def kernel(query, reference_points, value, value_spatial_shapes, Wv, bv, Wo,
           bo, Wa, ba, Wout, bout):
    del value_spatial_shapes
    f32 = jnp.float32

    qt = query.transpose(0, 2, 1)
    rpt = reference_points[:, :, 0, :].transpose(0, 2, 1)
    wo4 = Wo.reshape(H, P, 2, E)
    wox, woy = wo4[:, :, 0, :], wo4[:, :, 1, :]
    bo4 = bo.reshape(H, P, 2)
    box, boy = bo4[:, :, 0][:, :, None], bo4[:, :, 1][:, :, None]
    wa3 = Wa.reshape(H, P, E)
    ba3 = ba.reshape(H, P, 1)
    consts = jnp.asarray(_LEVEL_CONSTS)
    wvt = Wv.T
    bv2 = bv.reshape(1, E)
    wot = Wout.T
    bout2 = bout.reshape(1, E)

    nblk = LV // LV_BLK
    table = pl.pallas_call(
        _vproj_body,
        grid=(B, nblk),
        in_specs=[
            pl.BlockSpec((1, LV_BLK, E), lambda b, i: (b, i, 0)),
            pl.BlockSpec((E, E), lambda b, i: (0, 0)),
            pl.BlockSpec((1, E), lambda b, i: (0, 0)),
        ],
        out_specs=pl.BlockSpec((1, LV_BLK, E), lambda b, i: (b, i, 0)),
        out_shape=jax.ShapeDtypeStruct((B, LV, E), f32),
    )(value, wvt, bv2)

    idx, wts = pl.pallas_call(
        _idx_body,
        grid=(B, H),
        in_specs=[
            pl.BlockSpec((1, E, LQ), lambda b, h: (b, 0, 0)),
            pl.BlockSpec((1, 4, LQ), lambda b, h: (b, 0, 0)),
            pl.BlockSpec((1, P, E), lambda b, h: (h, 0, 0)),
            pl.BlockSpec((1, P, 1), lambda b, h: (h, 0, 0)),
            pl.BlockSpec((1, P, E), lambda b, h: (h, 0, 0)),
            pl.BlockSpec((1, P, 1), lambda b, h: (h, 0, 0)),
            pl.BlockSpec((1, P, E), lambda b, h: (h, 0, 0)),
            pl.BlockSpec((1, P, 1), lambda b, h: (h, 0, 0)),
            pl.BlockSpec((P, 3), lambda b, h: (0, 0)),
        ],
        out_specs=[
            pl.BlockSpec((1, 1, NCORN, LQ), lambda b, h: (b, h, 0, 0)),
            pl.BlockSpec((1, 1, NCORN, LQ), lambda b, h: (b, h, 0, 0)),
        ],
        out_shape=[
            jax.ShapeDtypeStruct((B, H, NCORN, LQ), jnp.int32),
            jax.ShapeDtypeStruct((B, H, NCORN, LQ), f32),
        ],
    )(qt, rpt, wa3, ba3, wox, box, woy, boy, consts)

    table_rows = table.reshape(B * LV * H, HD)
    idx5 = idx.reshape(B, H, 4, P, LQ).transpose(0, 1, 4, 3, 2)
    idx_flat = idx5.reshape(NW, NCHUNK, CHUNK)
    w5 = wts.reshape(B, H, 4, P, LQ).transpose(0, 1, 2, 4, 3)
    mesh = plsc.VectorSubcoreMesh(core_axis_name="c", subcore_axis_name="s")
    gathered = pl.kernel(
        _gather_body,
        out_type=jax.ShapeDtypeStruct((N4, HD), f32),
        mesh=mesh,
        scratch_types=[
            pltpu.VMEM((NCHUNK, CHUNK), jnp.int32),
            pltpu.VMEM((CHUNK, HD), f32),
            pltpu.VMEM((CHUNK, HD), f32),
            pltpu.SemaphoreType.DMA,
            pltpu.SemaphoreType.DMA,
        ],
        compiler_params=pltpu.CompilerParams(use_tc_tiling_on_sc=False),
    )(table_rows, idx_flat)

    g5 = gathered.reshape(B, H, LQ, P, 4 * HD)
    wot3 = wot.reshape(H, HD, E)
    out = pl.pallas_call(
        _reduce_body,
        grid=(B, H),
        in_specs=[
            pl.BlockSpec((1, 1, LQ, P, 4 * HD), lambda b, h: (b, h, 0, 0, 0)),
            pl.BlockSpec((1, 1, 4, LQ, P), lambda b, h: (b, h, 0, 0, 0)),
            pl.BlockSpec((1, HD, E), lambda b, h: (h, 0, 0)),
            pl.BlockSpec((1, E), lambda b, h: (0, 0)),
        ],
        out_specs=pl.BlockSpec((1, LQ, E), lambda b, h: (b, 0, 0)),
        out_shape=jax.ShapeDtypeStruct((B, LQ, E), f32),
    )(g5, w5, wot3, bout2)
    return out
